# baseline (speedup 1.0000x reference)
import functools

import jax
import jax.numpy as jnp
from jax import lax
from jax.experimental import pallas as pl
from jax.experimental.pallas import tpu as pltpu
from jax.experimental.pallas import tpu_sc as plsc

NUM_EMBEDDINGS = 1000000
EMBEDDING_DIM = 64
BATCH = 16384
HIST = 50

_info = plsc.get_sparse_core_info()
_NC = _info.num_cores
_NS = _info.num_subcores
_NW = _NC * _NS
_B_PER_W = BATCH // _NW
_PAIRS = _B_PER_W // 2


def _sc_gather(ids, weights):
    mesh = plsc.VectorSubcoreMesh(core_axis_name="c", subcore_axis_name="s")

    @functools.partial(
        pl.kernel,
        mesh=mesh,
        out_type=jax.ShapeDtypeStruct((BATCH, HIST, EMBEDDING_DIM), jnp.float32),
        scratch_types=[
            pltpu.VMEM((_B_PER_W, HIST), jnp.int32),
            pltpu.VMEM((HIST, EMBEDDING_DIM), jnp.float32),
            pltpu.VMEM((HIST, EMBEDDING_DIM), jnp.float32),
            pltpu.SemaphoreType.DMA,
            pltpu.SemaphoreType.DMA,
            pltpu.SemaphoreType.DMA,
            pltpu.SemaphoreType.DMA,
            pltpu.SemaphoreType.DMA,
        ],
        compiler_params=pltpu.CompilerParams(use_tc_tiling_on_sc=False),
    )
    def body(ids_hbm, table_hbm, out_hbm, idxbuf, rows0, rows1,
             isem, gsem0, gsem1, osem0, osem1):
        wid = lax.axis_index("s") * _NC + lax.axis_index("c")
        b0 = wid * _B_PER_W

        def gather(b, rowsv, gsem):
            return pltpu.async_copy(table_hbm.at[idxbuf.at[b]], rowsv, gsem)

        def wait_gather(b, rowsv, gsem):
            pltpu.make_async_copy(
                table_hbm.at[idxbuf.at[b]], rowsv, gsem).wait()

        def put(b, rowsv, osem):
            return pltpu.async_copy(rowsv, out_hbm.at[b0 + b], osem)

        def wait_put(b, rowsv, osem):
            pltpu.make_async_copy(rowsv, out_hbm.at[b0 + b], osem).wait()

        pltpu.async_copy(
            ids_hbm.at[pl.ds(b0, _B_PER_W)], idxbuf, isem).wait()
        gather(0, rows0, gsem0)

        def step(p, carry):
            b = 2 * p

            @pl.when(p >= 1)
            def _():
                wait_put(b - 1, rows1, osem1)
            gather(b + 1, rows1, gsem1)

            wait_gather(b, rows0, gsem0)
            put(b, rows0, osem0)

            @pl.when(p + 1 < _PAIRS)
            def _():
                wait_put(b, rows0, osem0)
                gather(b + 2, rows0, gsem0)

            wait_gather(b + 1, rows1, gsem1)
            put(b + 1, rows1, osem1)
            return carry

        lax.fori_loop(0, _PAIRS, step, 0)
        wait_put(_B_PER_W - 2, rows0, osem0)
        wait_put(_B_PER_W - 1, rows1, osem1)

    return body(ids, weights)


def kernel(token_ids, weights):
    return _sc_gather(token_ids.astype(jnp.int32), weights)


# speedup vs baseline: 1.3031x; 1.3031x over previous
import functools

import jax
import jax.numpy as jnp
from jax import lax
from jax.experimental import pallas as pl
from jax.experimental.pallas import tpu as pltpu
from jax.experimental.pallas import tpu_sc as plsc

NUM_EMBEDDINGS = 1000000
EMBEDDING_DIM = 64
BATCH = 16384
HIST = 50

_info = plsc.get_sparse_core_info()
_NC = _info.num_cores
_NS = _info.num_subcores
_NW = _NC * _NS
_B_PER_W = BATCH // _NW
_PAIRS = _B_PER_W // 2


def _sc_gather(ids, weights):
    mesh = plsc.VectorSubcoreMesh(core_axis_name="c", subcore_axis_name="s")

    @functools.partial(
        pl.kernel,
        mesh=mesh,
        out_type=jax.ShapeDtypeStruct((BATCH, 56, 128), jnp.float32),
        scratch_types=[
            pltpu.VMEM((_B_PER_W, HIST), jnp.int32),
            pltpu.VMEM((HIST, EMBEDDING_DIM), jnp.float32),
            pltpu.VMEM((HIST, EMBEDDING_DIM), jnp.float32),
            pltpu.SemaphoreType.DMA,
            pltpu.SemaphoreType.DMA,
            pltpu.SemaphoreType.DMA,
            pltpu.SemaphoreType.DMA,
            pltpu.SemaphoreType.DMA,
        ],
        compiler_params=pltpu.CompilerParams(use_tc_tiling_on_sc=False),
    )
    def body(ids_hbm, table_hbm, out_hbm, idxbuf, rows0, rows1,
             isem, gsem0, gsem1, osem0, osem1):
        wid = lax.axis_index("s") * _NC + lax.axis_index("c")
        b0 = wid * _B_PER_W

        def gather(b, rowsv, gsem):
            return pltpu.async_copy(table_hbm.at[idxbuf.at[b]], rowsv, gsem)

        def wait_gather(b, rowsv, gsem):
            pltpu.make_async_copy(
                table_hbm.at[idxbuf.at[b]], rowsv, gsem).wait()

        def put(b, rowsv, osem):
            return pltpu.async_copy(
                rowsv,
                out_hbm.at[b0 + b, pl.ds(0, HIST), pl.ds(0, EMBEDDING_DIM)],
                osem)

        def wait_put(b, rowsv, osem):
            pltpu.make_async_copy(
                rowsv,
                out_hbm.at[b0 + b, pl.ds(0, HIST), pl.ds(0, EMBEDDING_DIM)],
                osem).wait()

        pltpu.async_copy(
            ids_hbm.at[pl.ds(b0, _B_PER_W)], idxbuf, isem).wait()
        gather(0, rows0, gsem0)

        def step(p, carry):
            b = 2 * p

            @pl.when(p >= 1)
            def _():
                wait_put(b - 1, rows1, osem1)
            gather(b + 1, rows1, gsem1)

            wait_gather(b, rows0, gsem0)
            put(b, rows0, osem0)

            @pl.when(p + 1 < _PAIRS)
            def _():
                wait_put(b, rows0, osem0)
                gather(b + 2, rows0, gsem0)

            wait_gather(b + 1, rows1, gsem1)
            put(b + 1, rows1, osem1)
            return carry

        lax.fori_loop(0, _PAIRS, step, 0)
        wait_put(_B_PER_W - 2, rows0, osem0)
        wait_put(_B_PER_W - 1, rows1, osem1)

    return body(ids, weights)


def kernel(token_ids, weights):
    padded = _sc_gather(token_ids.astype(jnp.int32), weights)
    return padded[:, :HIST, :EMBEDDING_DIM]


# speedup vs baseline: 1.4670x; 1.1258x over previous
import functools

import jax
import jax.numpy as jnp
from jax import lax
from jax.experimental import pallas as pl
from jax.experimental.pallas import tpu as pltpu
from jax.experimental.pallas import tpu_sc as plsc

NUM_EMBEDDINGS = 1000000
EMBEDDING_DIM = 64
BATCH = 16384
HIST = 50

_HPAD = 56
_DPAD = 128
_BB = 128

_info = plsc.get_sparse_core_info()
_NC = _info.num_cores
_NS = _info.num_subcores
_NW = _NC * _NS
_B_PER_W = BATCH // _NW
_BLK_PER_W = _B_PER_W // _BB
_NTASK = HIST * _BLK_PER_W
_PAIRS = _NTASK // 2


def _sc_gather(tds, weights):
    mesh = plsc.VectorSubcoreMesh(core_axis_name="c", subcore_axis_name="s")

    @functools.partial(
        pl.kernel,
        mesh=mesh,
        out_type=jax.ShapeDtypeStruct((BATCH, _HPAD, _DPAD), jnp.float32),
        scratch_types=[
            pltpu.VMEM((HIST, _B_PER_W), jnp.int32),
            pltpu.VMEM((_BB, EMBEDDING_DIM), jnp.float32),
            pltpu.VMEM((_BB, EMBEDDING_DIM), jnp.float32),
            pltpu.SemaphoreType.DMA,
            pltpu.SemaphoreType.DMA,
            pltpu.SemaphoreType.DMA,
            pltpu.SemaphoreType.DMA,
            pltpu.SemaphoreType.DMA,
        ],
        compiler_params=pltpu.CompilerParams(use_tc_tiling_on_sc=False),
    )
    def body(tds_hbm, table_hbm, out_hbm, idxbuf, rows0, rows1,
             isem, gsem0, gsem1, osem0, osem1):
        wid = lax.axis_index("s") * _NC + lax.axis_index("c")
        b0 = wid * _B_PER_W

        def gather(n, rowsv, gsem):
            h = n // _BLK_PER_W
            j = n % _BLK_PER_W
            return pltpu.async_copy(
                table_hbm.at[idxbuf.at[h, pl.ds(j * _BB, _BB)]], rowsv, gsem)

        def wait_gather(n, rowsv, gsem):
            h = n // _BLK_PER_W
            j = n % _BLK_PER_W
            pltpu.make_async_copy(
                table_hbm.at[idxbuf.at[h, pl.ds(j * _BB, _BB)]],
                rowsv, gsem).wait()

        def _dst(n):
            h = n // _BLK_PER_W
            j = n % _BLK_PER_W
            return out_hbm.at[
                pl.ds(b0 + j * _BB, _BB), h, pl.ds(0, EMBEDDING_DIM)]

        def put(n, rowsv, osem):
            return pltpu.async_copy(rowsv, _dst(n), osem)

        def wait_put(n, rowsv, osem):
            pltpu.make_async_copy(rowsv, _dst(n), osem).wait()

        pltpu.async_copy(
            tds_hbm.at[:, pl.ds(b0, _B_PER_W)], idxbuf, isem).wait()
        gather(0, rows0, gsem0)

        def step(p, carry):
            n = 2 * p

            @pl.when(p >= 1)
            def _():
                wait_put(n - 1, rows1, osem1)
            gather(n + 1, rows1, gsem1)

            wait_gather(n, rows0, gsem0)
            put(n, rows0, osem0)

            @pl.when(p + 1 < _PAIRS)
            def _():
                wait_put(n, rows0, osem0)
                gather(n + 2, rows0, gsem0)

            wait_gather(n + 1, rows1, gsem1)
            put(n + 1, rows1, osem1)
            return carry

        lax.fori_loop(0, _PAIRS, step, 0)
        wait_put(_NTASK - 2, rows0, osem0)
        wait_put(_NTASK - 1, rows1, osem1)

    return body(tds, weights)


def kernel(token_ids, weights):
    tds = token_ids.astype(jnp.int32).T
    padded = _sc_gather(tds, weights)
    return padded[:, :HIST, :EMBEDDING_DIM]
